# Initial kernel scaffold; baseline (speedup 1.0000x reference)
import jax
import jax.numpy as jnp
from jax.experimental import pallas as pl


def kernel(x, edge_index, W1l, b1, W1r, W2l, b2, W2r):
    raise NotImplementedError("write your pallas kernel here")


# speedup vs baseline: 5.2512x; 5.2512x over previous
import functools

import jax
import jax.numpy as jnp
from jax import lax
from jax.experimental import pallas as pl
from jax.experimental.pallas import tpu as pltpu
from jax.experimental.pallas import tpu_sc as plsc

NC = 2
NS = 16
NW = NC * NS


def _make_agg(n_nodes, n_edges, d, with_counts):
    epw = n_edges // NW
    B = 80
    nch = epw // B
    assert nch * B == epw
    rpt = n_nodes // NS
    RB = 80
    nrb = rpt // RB
    assert nrb * RB == rpt

    out_type = [jax.ShapeDtypeStruct((NC, n_nodes, d), jnp.float32)]
    scratch = [
        pltpu.VMEM((B,), jnp.int32),
        pltpu.VMEM((B,), jnp.int32),
        pltpu.VMEM((RB, d), jnp.float32),
        pltpu.VMEM_SHARED((n_nodes, d), jnp.float32),
        pltpu.SemaphoreType.DMA,
    ]
    if with_counts:
        out_type.append(jax.ShapeDtypeStruct((NC, n_nodes), jnp.float32))
        scratch += [
            pltpu.VMEM((B,), jnp.float32),
            pltpu.VMEM((rpt,), jnp.float32),
            pltpu.VMEM_SHARED((n_nodes,), jnp.float32),
        ]

    mesh = plsc.VectorSubcoreMesh(core_axis_name="c", subcore_axis_name="s")

    @functools.partial(pl.kernel, out_type=out_type, mesh=mesh,
                       scratch_types=scratch)
    def agg(*refs):
        if with_counts:
            (y_hbm, src_hbm, dst_hbm, part_hbm, cnt_hbm,
             idx_s, idx_d, rows, acc, sem, ones, cbuf, cacc) = refs
        else:
            (y_hbm, src_hbm, dst_hbm, part_hbm,
             idx_s, idx_d, rows, acc, sem) = refs
        cid = lax.axis_index("c")
        sid = lax.axis_index("s")
        wid = sid * NC + cid
        ebase = wid * epw
        rbase = sid * rpt

        def zrow(i, carry):
            for j in range(d // 16):
                rows[i, pl.ds(j * 16, 16)] = jnp.zeros((16,), jnp.float32)
            return carry
        lax.fori_loop(0, RB, zrow, 0)
        for j in range(nrb):
            pltpu.sync_copy(rows, acc.at[pl.ds(rbase + j * RB, RB)])
        if with_counts:
            def zcnt(i, carry):
                cbuf[pl.ds(i * 16, 16)] = jnp.zeros((16,), jnp.float32)
                return carry
            lax.fori_loop(0, rpt // 16, zcnt, 0)
            pltpu.sync_copy(cbuf, cacc.at[pl.ds(rbase, rpt)])
            for k in range(B // 16):
                ones[pl.ds(k * 16, 16)] = jnp.ones((16,), jnp.float32)
        plsc.subcore_barrier()

        def step(k, carry):
            eb = ebase + k * B
            pltpu.sync_copy(src_hbm.at[pl.ds(eb, B)], idx_s)
            pltpu.sync_copy(dst_hbm.at[pl.ds(eb, B)], idx_d)
            pltpu.async_copy(y_hbm.at[idx_s], rows, sem).wait()
            pltpu.sync_copy(rows, acc.at[idx_d], add=True)
            if with_counts:
                pltpu.sync_copy(ones, cacc.at[idx_d], add=True)
            return carry
        lax.fori_loop(0, nch, step, 0)
        plsc.subcore_barrier()

        for j in range(nrb):
            r0 = rbase + j * RB
            pltpu.sync_copy(acc.at[pl.ds(r0, RB)], rows)
            pltpu.sync_copy(rows, part_hbm.at[cid, pl.ds(r0, RB)])
        if with_counts:
            pltpu.sync_copy(cacc.at[pl.ds(rbase, rpt)], cbuf)
            pltpu.sync_copy(cbuf, cnt_hbm.at[cid, pl.ds(rbase, rpt)])

    return agg


def _mm1_body(x_ref, wl_ref, wr_ref, b_ref, yl_ref, yr_ref):
    xb = x_ref[...]
    dn = (((1,), (1,)), ((), ()))
    yl_ref[...] = lax.dot_general(xb, wl_ref[...], dn,
                                  preferred_element_type=jnp.float32)
    yr_ref[...] = lax.dot_general(xb, wr_ref[...], dn,
                                  preferred_element_type=jnp.float32) + b_ref[...]


def _mm2_body(p_ref, c_ref, yr_ref, wl_ref, wr_ref, b_ref, yl2_ref, yr2_ref):
    psum = p_ref[0] + p_ref[1]
    c = c_ref[0] + c_ref[1]
    inv = 1.0 / jnp.maximum(c, 1.0)
    h = jnp.maximum(psum * inv + yr_ref[...], 0.0)
    dn = (((1,), (1,)), ((), ()))
    yl2_ref[...] = lax.dot_general(h, wl_ref[...], dn,
                                   preferred_element_type=jnp.float32)
    yr2_ref[...] = lax.dot_general(h, wr_ref[...], dn,
                                   preferred_element_type=jnp.float32) + b_ref[...]


def _fin_body(p_ref, c_ref, yr_ref, o_ref):
    psum = p_ref[0] + p_ref[1]
    c = c_ref[0] + c_ref[1]
    inv = 1.0 / jnp.maximum(c, 1.0)
    o_ref[...] = psum * inv + yr_ref[...]


def kernel(x, edge_index, W1l, b1, W1r, W2l, b2, W2r):
    n, d_in = x.shape
    e = edge_index.shape[1]
    d_hid = W1l.shape[0]
    n_cls = W2l.shape[0]
    d2 = 128
    n_pad = ((n + NS * 16 - 1) // (NS * 16)) * (NS * 16)

    src = edge_index[0].astype(jnp.int32)
    dst = edge_index[1].astype(jnp.int32)

    R = 400
    grid = (n // R,)

    yl1, yr1 = pl.pallas_call(
        _mm1_body,
        grid=grid,
        in_specs=[
            pl.BlockSpec((R, d_in), lambda g: (g, 0)),
            pl.BlockSpec((d_hid, d_in), lambda g: (0, 0)),
            pl.BlockSpec((d_hid, d_in), lambda g: (0, 0)),
            pl.BlockSpec((1, d_hid), lambda g: (0, 0)),
        ],
        out_specs=[pl.BlockSpec((R, d_hid), lambda g: (g, 0)),
                   pl.BlockSpec((R, d_hid), lambda g: (g, 0))],
        out_shape=[jax.ShapeDtypeStruct((n, d_hid), jnp.float32),
                   jax.ShapeDtypeStruct((n, d_hid), jnp.float32)],
    )(x, W1l, W1r, b1[None])

    part1, cnt = _make_agg(n_pad, e, d_hid, True)(yl1, src, dst)
    cnt3 = cnt[:, :, None]

    W2lp = jnp.zeros((d2, d_hid), jnp.float32).at[:n_cls].set(W2l)
    W2rp = jnp.zeros((d2, d_hid), jnp.float32).at[:n_cls].set(W2r)
    b2p = jnp.zeros((1, d2), jnp.float32).at[0, :n_cls].set(b2)
    yl2, yr2 = pl.pallas_call(
        _mm2_body,
        grid=grid,
        in_specs=[
            pl.BlockSpec((NC, R, d_hid), lambda g: (0, g, 0)),
            pl.BlockSpec((NC, R, 1), lambda g: (0, g, 0)),
            pl.BlockSpec((R, d_hid), lambda g: (g, 0)),
            pl.BlockSpec((d2, d_hid), lambda g: (0, 0)),
            pl.BlockSpec((d2, d_hid), lambda g: (0, 0)),
            pl.BlockSpec((1, d2), lambda g: (0, 0)),
        ],
        out_specs=[pl.BlockSpec((R, d2), lambda g: (g, 0)),
                   pl.BlockSpec((R, d2), lambda g: (g, 0))],
        out_shape=[jax.ShapeDtypeStruct((n, d2), jnp.float32),
                   jax.ShapeDtypeStruct((n, d2), jnp.float32)],
    )(part1, cnt3, yr1, W2lp, W2rp, b2p)

    (part2,) = _make_agg(n_pad, e, d2, False)(yl2, src, dst)

    out = pl.pallas_call(
        _fin_body,
        grid=grid,
        in_specs=[
            pl.BlockSpec((NC, R, d2), lambda g: (0, g, 0)),
            pl.BlockSpec((NC, R, 1), lambda g: (0, g, 0)),
            pl.BlockSpec((R, d2), lambda g: (g, 0)),
        ],
        out_specs=pl.BlockSpec((R, d2), lambda g: (g, 0)),
        out_shape=jax.ShapeDtypeStruct((n, d2), jnp.float32),
    )(part2, cnt3, yr2)

    return out[:, :n_cls]
